# initial kernel scaffold (unmeasured)
import jax
import jax.numpy as jnp
from jax import lax
from jax.experimental import pallas as pl
from jax.experimental.pallas import tpu as pltpu


def kernel(
    x,
):
    def body(*refs):
        pass

    out_shape = jax.ShapeDtypeStruct(..., jnp.float32)
    return pl.pallas_call(body, out_shape=out_shape)(...)


# baseline (device time: 1172762 ns/iter reference)
import jax
import jax.numpy as jnp
from jax import lax
from jax.experimental import pallas as pl
from jax.experimental.pallas import tpu as pltpu

N_Z = 4
PLUS, MINUS = 0, 1


def kernel(x):
    m_per, n = x.shape
    half = m_per // 2
    n_hops = N_Z - 1

    def body(x_ref, out_ref, local_sem, send_sems, recv_sems):
        my_x = lax.axis_index("x")
        my_y = lax.axis_index("y")
        my_z = lax.axis_index("z")
        right = (my_z + 1) % N_Z
        left = (my_z + N_Z - 1) % N_Z

        barrier_sem = pltpu.get_barrier_semaphore()
        for nbr in (left, right):
            pl.semaphore_signal(
                barrier_sem,
                inc=1,
                device_id=(my_x, my_y, nbr),
                device_id_type=pl.DeviceIdType.MESH,
            )
        pl.semaphore_wait(barrier_sem, 2)

        local_copy = pltpu.make_async_copy(
            x_ref, out_ref.at[pl.ds(my_z * m_per, m_per)], local_sem
        )
        local_copy.start()

        def send_plus(h):
            origin = (my_z - h + N_Z) % N_Z
            src = x_ref.at[pl.ds(0, half)] if h == 0 else out_ref.at[
                pl.ds(origin * m_per, half)
            ]
            return pltpu.make_async_remote_copy(
                src_ref=src,
                dst_ref=out_ref.at[pl.ds(origin * m_per, half)],
                send_sem=send_sems.at[PLUS, h],
                recv_sem=recv_sems.at[PLUS, h],
                device_id=(my_x, my_y, right),
                device_id_type=pl.DeviceIdType.MESH,
            )

        def send_minus(h):
            origin = (my_z + h) % N_Z
            src = x_ref.at[pl.ds(half, half)] if h == 0 else out_ref.at[
                pl.ds(origin * m_per + half, half)
            ]
            return pltpu.make_async_remote_copy(
                src_ref=src,
                dst_ref=out_ref.at[pl.ds(origin * m_per + half, half)],
                send_sem=send_sems.at[MINUS, h],
                recv_sem=recv_sems.at[MINUS, h],
                device_id=(my_x, my_y, left),
                device_id_type=pl.DeviceIdType.MESH,
            )

        def recv_plus(h):
            origin = (my_z - h - 1 + N_Z) % N_Z
            return pltpu.make_async_remote_copy(
                src_ref=x_ref.at[pl.ds(0, half)],
                dst_ref=out_ref.at[pl.ds(origin * m_per, half)],
                send_sem=send_sems.at[PLUS, h],
                recv_sem=recv_sems.at[PLUS, h],
                device_id=(my_x, my_y, right),
                device_id_type=pl.DeviceIdType.MESH,
            )

        def recv_minus(h):
            origin = (my_z + h + 1) % N_Z
            return pltpu.make_async_remote_copy(
                src_ref=x_ref.at[pl.ds(half, half)],
                dst_ref=out_ref.at[pl.ds(origin * m_per + half, half)],
                send_sem=send_sems.at[MINUS, h],
                recv_sem=recv_sems.at[MINUS, h],
                device_id=(my_x, my_y, left),
                device_id_type=pl.DeviceIdType.MESH,
            )

        sends = []
        sp = send_plus(0)
        sp.start()
        sm = send_minus(0)
        sm.start()
        sends += [sp, sm]
        for h in range(1, n_hops):
            recv_plus(h - 1).wait_recv()
            sp = send_plus(h)
            sp.start()
            recv_minus(h - 1).wait_recv()
            sm = send_minus(h)
            sm.start()
            sends += [sp, sm]
        recv_plus(n_hops - 1).wait_recv()
        recv_minus(n_hops - 1).wait_recv()
        for s in sends:
            s.wait_send()
        local_copy.wait()

    return pl.pallas_call(
        body,
        out_shape=jax.ShapeDtypeStruct((N_Z * m_per, n), x.dtype),
        in_specs=[pl.BlockSpec(memory_space=pl.ANY)],
        out_specs=pl.BlockSpec(memory_space=pl.ANY),
        scratch_shapes=[
            pltpu.SemaphoreType.DMA,
            pltpu.SemaphoreType.DMA((2, 3)),
            pltpu.SemaphoreType.DMA((2, 3)),
        ],
        compiler_params=pltpu.CompilerParams(collective_id=0),
    )(x)


# device time: 1110821 ns/iter; 1.0558x vs baseline; 1.0558x over previous
import jax
import jax.numpy as jnp
from jax import lax
from jax.experimental import pallas as pl
from jax.experimental.pallas import tpu as pltpu

N_Z = 4
NB = 4
NMSG = (N_Z - 1) * NB
R, L, YR, YL = 0, 1, 2, 3


def kernel(x):
    m_per, n = x.shape
    half = m_per // 2
    blk = half // NB

    def body(x_ref, out_ref, local_sem, send_sems, recv_sems):
        my_x = lax.axis_index("x")
        my_y = lax.axis_index("y")
        my_z = lax.axis_index("z")
        right_z = jnp.minimum(my_z + 1, N_Z - 1)
        left_z = jnp.maximum(my_z - 1, 0)
        y_off = my_y * half
        py_off = (1 - my_y) * half

        barrier_sem = pltpu.get_barrier_semaphore()
        for tgt in (
            (my_x, my_y, left_z),
            (my_x, my_y, right_z),
            (my_x, 1 - my_y, my_z),
        ):
            pl.semaphore_signal(
                barrier_sem,
                inc=1,
                device_id=tgt,
                device_id_type=pl.DeviceIdType.MESH,
            )
        pl.semaphore_wait(barrier_sem, 3)

        local_copy = pltpu.make_async_copy(
            x_ref, out_ref.at[pl.ds(my_z * m_per, m_per)], local_sem
        )
        local_copy.start()


        def r_send(k):
            s, b = divmod(k, NB)
            origin = jnp.maximum(my_z - s, 0)
            row = origin * m_per + y_off + b * blk
            src = (
                x_ref.at[pl.ds(y_off + b * blk, blk)]
                if s == 0
                else out_ref.at[pl.ds(row, blk)]
            )
            guard = (my_z < N_Z - 1) & (my_z >= s)
            return guard, pltpu.make_async_remote_copy(
                src_ref=src,
                dst_ref=out_ref.at[pl.ds(row, blk)],
                send_sem=send_sems.at[R, k],
                recv_sem=recv_sems.at[R, k],
                device_id=(my_x, my_y, right_z),
                device_id_type=pl.DeviceIdType.MESH,
            )

        def l_send(k):
            s, b = divmod(k, NB)
            origin = jnp.minimum(my_z + s, N_Z - 1)
            row = origin * m_per + y_off + b * blk
            src = (
                x_ref.at[pl.ds(y_off + b * blk, blk)]
                if s == 0
                else out_ref.at[pl.ds(row, blk)]
            )
            guard = (my_z >= 1) & (my_z + s <= N_Z - 1)
            return guard, pltpu.make_async_remote_copy(
                src_ref=src,
                dst_ref=out_ref.at[pl.ds(row, blk)],
                send_sem=send_sems.at[L, k],
                recv_sem=recv_sems.at[L, k],
                device_id=(my_x, my_y, left_z),
                device_id_type=pl.DeviceIdType.MESH,
            )

        def r_recv(k):
            s, b = divmod(k, NB)
            origin = jnp.maximum(my_z - 1 - s, 0)
            row = origin * m_per + y_off + b * blk
            guard = my_z >= s + 1
            return guard, origin, pltpu.make_async_remote_copy(
                src_ref=out_ref.at[pl.ds(row, blk)],
                dst_ref=out_ref.at[pl.ds(row, blk)],
                send_sem=send_sems.at[R, k],
                recv_sem=recv_sems.at[R, k],
                device_id=(my_x, my_y, left_z),
                device_id_type=pl.DeviceIdType.MESH,
            )

        def l_recv(k):
            s, b = divmod(k, NB)
            origin = jnp.minimum(my_z + 1 + s, N_Z - 1)
            row = origin * m_per + y_off + b * blk
            guard = my_z + 1 + s <= N_Z - 1
            return guard, origin, pltpu.make_async_remote_copy(
                src_ref=out_ref.at[pl.ds(row, blk)],
                dst_ref=out_ref.at[pl.ds(row, blk)],
                send_sem=send_sems.at[L, k],
                recv_sem=recv_sems.at[L, k],
                device_id=(my_x, my_y, right_z),
                device_id_type=pl.DeviceIdType.MESH,
            )

        def y_fwd(stream, k, origin):
            s, b = divmod(k, NB)
            row = origin * m_per + y_off + b * blk
            return pltpu.make_async_remote_copy(
                src_ref=out_ref.at[pl.ds(row, blk)],
                dst_ref=out_ref.at[pl.ds(row, blk)],
                send_sem=send_sems.at[stream, k],
                recv_sem=recv_sems.at[stream, k],
                device_id=(my_x, 1 - my_y, my_z),
                device_id_type=pl.DeviceIdType.MESH,
            )

        def y_recv(stream, k):
            s, b = divmod(k, NB)
            if stream == YR:
                origin = jnp.maximum(my_z - 1 - s, 0)
                guard = my_z >= s + 1
            else:
                origin = jnp.minimum(my_z + 1 + s, N_Z - 1)
                guard = my_z + 1 + s <= N_Z - 1
            row = origin * m_per + py_off + b * blk
            return guard, pltpu.make_async_remote_copy(
                src_ref=out_ref.at[pl.ds(row, blk)],
                dst_ref=out_ref.at[pl.ds(row, blk)],
                send_sem=send_sems.at[stream, k],
                recv_sem=recv_sems.at[stream, k],
                device_id=(my_x, 1 - my_y, my_z),
                device_id_type=pl.DeviceIdType.MESH,
            )

        started = []

        def start(guard, desc):
            pl.when(guard)(desc.start)
            started.append((guard, desc))

        for k in range(NB):
            start(*r_send(k))
            start(*l_send(k))

        for k in range(NMSG):
            rg, rorigin, rrecv = r_recv(k)
            pl.when(rg)(rrecv.wait_recv)
            if k + NB < NMSG:
                start(*r_send(k + NB))
            yf = y_fwd(YR, k, rorigin)
            pl.when(rg)(yf.start)
            started.append((rg, yf))

            lg, lorigin, lrecv = l_recv(k)
            pl.when(lg)(lrecv.wait_recv)
            if k + NB < NMSG:
                start(*l_send(k + NB))
            yf = y_fwd(YL, k, lorigin)
            pl.when(lg)(yf.start)
            started.append((lg, yf))

        for k in range(NMSG):
            g, d = y_recv(YR, k)
            pl.when(g)(d.wait_recv)
            g, d = y_recv(YL, k)
            pl.when(g)(d.wait_recv)
        for g, d in started:
            pl.when(g)(d.wait_send)
        local_copy.wait()

    return pl.pallas_call(
        body,
        out_shape=jax.ShapeDtypeStruct((N_Z * m_per, n), x.dtype),
        in_specs=[pl.BlockSpec(memory_space=pl.ANY)],
        out_specs=pl.BlockSpec(memory_space=pl.ANY),
        scratch_shapes=[
            pltpu.SemaphoreType.DMA,
            pltpu.SemaphoreType.DMA((4, NMSG)),
            pltpu.SemaphoreType.DMA((4, NMSG)),
        ],
        compiler_params=pltpu.CompilerParams(collective_id=0),
    )(x)
